# baseline (speedup 1.0000x reference)
import jax
import jax.numpy as jnp
from jax.experimental import pallas as pl

N, C, K = 1024, 12, 3
D = 5000
BLOCK_N = 8


def _rotl(v, d):
    return jax.lax.shift_left(v, jnp.uint32(d)) | jax.lax.shift_right_logical(
        v, jnp.uint32(32 - d))


def _threefry_bits(j):
    k1 = jnp.uint32(0)
    k2 = jnp.uint32(42)
    ks = (k1, k2, k1 ^ k2 ^ jnp.uint32(0x1BD11BDA))
    rot = ((13, 15, 26, 6), (17, 29, 16, 24))
    x0 = jnp.zeros_like(j) + ks[0]
    x1 = j + ks[1]
    for i in range(5):
        for r in rot[i % 2]:
            x0 = x0 + x1
            x1 = _rotl(x1, r)
            x1 = x1 ^ x0
        x0 = x0 + ks[(i + 1) % 3]
        x1 = x1 + ks[(i + 2) % 3] + jnp.uint32(i + 1)
    return x0 ^ x1


def _keep_mask(i, rb):
    rr = jax.lax.broadcasted_iota(jnp.uint32, (rb, C), 0)
    cc = jax.lax.broadcasted_iota(jnp.uint32, (rb, C), 1)
    n = (i * rb).astype(jnp.uint32) + rr
    bits = _threefry_bits(n * jnp.uint32(C) + cc)
    v = jax.lax.shift_right_logical(bits, jnp.uint32(9)).astype(jnp.int32)
    cc_i = cc.astype(jnp.int32)
    rank = jnp.zeros((rb, C), dtype=jnp.int32)
    for cp in range(C):
        vc = v[:, cp:cp + 1]
        lt = (vc < v) | ((vc == v) & (cp < cc_i))
        rank = rank + lt.astype(jnp.int32)
    return (rank >= K).astype(jnp.float32)


def _body(x_ref, o_ref):
    o_ref[...] = x_ref[...]


def kernel(x):
    return pl.pallas_call(
        _body,
        grid=(N // BLOCK_N,),
        in_specs=[pl.BlockSpec((BLOCK_N, C, D), lambda i: (i, 0, 0))],
        out_specs=pl.BlockSpec((BLOCK_N, C, D), lambda i: (i, 0, 0)),
        out_shape=jax.ShapeDtypeStruct((N, C, D), x.dtype),
    )(x)


# speedup vs baseline: 5.1764x; 5.1764x over previous
import jax
import jax.numpy as jnp
from jax.experimental import pallas as pl
from jax.experimental.pallas import tpu as pltpu

N, C, K = 1024, 12, 3
D = 5000
BLOCK_D = 1000


def _rotl(v, d):
    return jax.lax.shift_left(v, jnp.uint32(d)) | jax.lax.shift_right_logical(
        v, jnp.uint32(32 - d))


def _threefry_bits(j):
    k1 = jnp.uint32(0)
    k2 = jnp.uint32(42)
    ks = (k1, k2, k1 ^ k2 ^ jnp.uint32(0x1BD11BDA))
    rot = ((13, 15, 26, 6), (17, 29, 16, 24))
    x0 = jnp.zeros_like(j) + ks[0]
    x1 = j + ks[1]
    for i in range(5):
        for r in rot[i % 2]:
            x0 = x0 + x1
            x1 = _rotl(x1, r)
            x1 = x1 ^ x0
        x0 = x0 + ks[(i + 1) % 3]
        x1 = x1 + ks[(i + 2) % 3] + jnp.uint32(i + 1)
    return x0 ^ x1


def _keep_all():
    cc = jax.lax.broadcasted_iota(jnp.uint32, (C, N), 0)
    nn = jax.lax.broadcasted_iota(jnp.uint32, (C, N), 1)
    bits = _threefry_bits(nn * jnp.uint32(C) + cc)
    v = jax.lax.shift_right_logical(bits, jnp.uint32(9)).astype(jnp.int32)
    cc_i = cc.astype(jnp.int32)
    rank = jnp.zeros((C, N), dtype=jnp.int32)
    for cp in range(C):
        vp = v[cp:cp + 1, :]
        lt = (vp < v) | ((vp == v) & (cp < cc_i))
        rank = rank + lt.astype(jnp.int32)
    return (rank >= K).astype(jnp.float32)


def _body(x_ref, o_ref, keep_ref):
    i = pl.program_id(0)
    j = pl.program_id(1)

    @pl.when((i == 0) & (j == 0))
    def _():
        keep_ref[...] = _keep_all()

    krow = keep_ref[pl.ds(i, 1), :]
    o_ref[...] = x_ref[...] * krow[None]


def kernel(x):
    xt = jax.lax.transpose(x, (1, 2, 0))
    ot = pl.pallas_call(
        _body,
        grid=(C, D // BLOCK_D),
        in_specs=[pl.BlockSpec((1, BLOCK_D, N), lambda i, j: (i, j, 0))],
        out_specs=pl.BlockSpec((1, BLOCK_D, N), lambda i, j: (i, j, 0)),
        out_shape=jax.ShapeDtypeStruct((C, D, N), x.dtype),
        scratch_shapes=[pltpu.VMEM((C, N), jnp.float32)],
    )(xt)
    return jax.lax.transpose(ot, (2, 0, 1))
